# Initial kernel scaffold; baseline (speedup 1.0000x reference)
import jax
import jax.numpy as jnp
from jax.experimental import pallas as pl


def kernel(x, edge_index, edge_weights, batch, W1, b1, W2, b2, eps, Wl, bl):
    raise NotImplementedError("write your pallas kernel here")


# speedup vs baseline: 3.5199x; 3.5199x over previous
import functools

import jax
import jax.numpy as jnp
from jax import lax
from jax.experimental import pallas as pl
from jax.experimental.pallas import tpu as pltpu
from jax.experimental.pallas import tpu_sc as plsc

N, E, D, HID, OUT, G = 10000, 320000, 128, 128, 128, 64
NPAD = 10240
EB = 128
NB = E // EB
NW = 32
BPW = -(-NB // NW)
LANES = 16


def _sc_edge_body(x_hbm, src_hbm, dst_hbm, ew_hbm, out_hbm,
                  rows_v, ew_v, srci_v, dsti_v, agg_sh):
    cid = lax.axis_index("c")
    sid = lax.axis_index("s")
    w = sid * 2 + cid

    def zrow(i, _):
        for j in range(D // LANES):
            rows_v[i, pl.ds(j * LANES, LANES)] = jnp.zeros((LANES,), jnp.float32)
        return 0
    lax.fori_loop(0, EB, zrow, 0)
    for k in range(5):
        r0 = (sid * 5 + k) * 128
        pltpu.sync_copy(rows_v, agg_sh.at[pl.ds(r0, 128)])
    plsc.subcore_barrier()

    def edge_block(b, _):
        blk = w + b * NW

        @pl.when(blk < NB)
        def _():
            base = pl.multiple_of(blk * EB, EB)
            pltpu.sync_copy(src_hbm.at[pl.ds(base, EB)], srci_v)
            pltpu.sync_copy(dst_hbm.at[pl.ds(base, EB)], dsti_v)
            pltpu.sync_copy(x_hbm.at[srci_v], rows_v)
            pltpu.sync_copy(ew_hbm.at[pl.ds(base, EB)], ew_v)

            def comp(i, _2):
                for j in range(D // LANES):
                    sl = (i, pl.ds(j * LANES, LANES))
                    rows_v[sl] = jnp.maximum(rows_v[sl] + ew_v[sl], 0.0)
                return 0
            lax.fori_loop(0, EB, comp, 0)
            pltpu.sync_copy(rows_v, agg_sh.at[dsti_v], add=True)
        return 0
    lax.fori_loop(0, BPW, edge_block, 0)
    plsc.subcore_barrier()

    for k in range(5):
        r0 = (sid * 5 + k) * 128
        pltpu.sync_copy(agg_sh.at[pl.ds(r0, 128)], out_hbm.at[cid, pl.ds(r0, 128)])


_sc_edge = pl.kernel(
    _sc_edge_body,
    out_type=jax.ShapeDtypeStruct((2, NPAD, D), jnp.float32),
    mesh=plsc.VectorSubcoreMesh(core_axis_name="c", subcore_axis_name="s",
                                num_cores=2, num_subcores=16),
    scratch_types=[
        pltpu.VMEM((EB, D), jnp.float32),
        pltpu.VMEM((EB, D), jnp.float32),
        pltpu.VMEM((EB,), jnp.int32),
        pltpu.VMEM((EB,), jnp.int32),
        pltpu.VMEM_SHARED((NPAD, D), jnp.float32),
    ],
)


def _tc_dense_body(x_ref, agg_ref, bid_ref, W1_ref, b1_ref, W2_ref, b2_ref,
                   eps_ref, Wl_ref, bl_ref, out_ref, h_ref, pooled_ref):
    h0 = (1.0 + eps_ref[...]) * x_ref[...] + agg_ref[0, :N, :] + agg_ref[1, :N, :]
    t = jnp.maximum(
        jnp.dot(h0, W1_ref[...], preferred_element_type=jnp.float32) + b1_ref[...],
        0.0)
    h_ref[...] = jnp.dot(t, W2_ref[...], preferred_element_type=jnp.float32) + b2_ref[...]

    def pool_g(g, _):
        sel = jnp.where(bid_ref[...] == g, h_ref[...], -3.4e38)
        pooled_ref[pl.ds(g, 1), :] = jnp.max(sel, axis=0)[None, :]
        return 0
    lax.fori_loop(0, G, pool_g, 0)

    out_ref[...] = (jnp.dot(pooled_ref[...], Wl_ref[...],
                            preferred_element_type=jnp.float32)
                    + bl_ref[...])


_tc_dense = pl.pallas_call(
    _tc_dense_body,
    out_shape=jax.ShapeDtypeStruct((G, OUT), jnp.float32),
    scratch_shapes=[pltpu.VMEM((N, HID), jnp.float32),
                    pltpu.VMEM((G, HID), jnp.float32)],
)


def kernel(x, edge_index, edge_weights, batch, W1, b1, W2, b2, eps, Wl, bl):
    src = edge_index[0]
    dst = edge_index[1]
    agg = _sc_edge(x, src, dst, edge_weights)
    out = _tc_dense(x, agg, batch.reshape(N, 1),
                    W1, b1.reshape(1, -1), W2, b2.reshape(1, -1),
                    eps.reshape(1, 1), Wl, bl.reshape(1, -1))
    return out


# speedup vs baseline: 5.0277x; 1.4284x over previous
import jax
import jax.numpy as jnp
from jax import lax
from jax.experimental import pallas as pl
from jax.experimental.pallas import tpu as pltpu
from jax.experimental.pallas import tpu_sc as plsc

N, E, D, HID, OUT, G = 10000, 320000, 128, 128, 128, 64
NPAD = 10240
EB = 64
NB = E // EB
NW = 32
BPW = 158
EPAD = NW * BPW * EB
LANES = 16


def _sc_edge_body(x_hbm, src_hbm, dst_hbm, ew_hbm, out_hbm,
                  rows_v, ew_v, srci, dsti, agg_sh,
                  sem_is, sem_id, sem_g, sem_e, sem_s):
    cid = lax.axis_index("c")
    sid = lax.axis_index("s")
    w = sid * 2 + cid

    def zrow(i, _):
        for j in range(D // LANES):
            rows_v[0, i, pl.ds(j * LANES, LANES)] = jnp.zeros((LANES,), jnp.float32)
        return 0
    lax.fori_loop(0, EB, zrow, 0)
    for k in range(10):
        r0 = (sid * 10 + k) * EB
        pltpu.sync_copy(rows_v.at[0], agg_sh.at[pl.ds(r0, EB)])
    plsc.subcore_barrier()

    def issue_idx(blk, s2, s4):
        base = pl.multiple_of(blk * EB, EB)
        pltpu.async_copy(src_hbm.at[pl.ds(base, EB)], srci.at[s2], sem_is.at[s2])
        pltpu.async_copy(dst_hbm.at[pl.ds(base, EB)], dsti.at[s4], sem_id.at[s2])

    def wait_idx(s2):
        pltpu.make_async_copy(src_hbm.at[pl.ds(0, EB)], srci.at[s2], sem_is.at[s2]).wait()
        pltpu.make_async_copy(dst_hbm.at[pl.ds(0, EB)], dsti.at[0], sem_id.at[s2]).wait()

    def issue_main(blk, s2):
        pltpu.async_copy(x_hbm.at[srci.at[s2]], rows_v.at[s2], sem_g.at[s2])
        base_ew = pl.multiple_of(jnp.minimum(blk, NB - 1) * EB, EB)
        pltpu.async_copy(ew_hbm.at[pl.ds(base_ew, EB)], ew_v.at[s2], sem_e.at[s2])

    def wait_main(s2):
        pltpu.make_async_copy(x_hbm.at[srci.at[s2]], rows_v.at[s2], sem_g.at[s2]).wait()
        pltpu.make_async_copy(ew_hbm.at[pl.ds(0, EB)], ew_v.at[s2], sem_e.at[s2]).wait()

    def wait_scatter(s2, s4):
        pltpu.make_async_copy(rows_v.at[s2], agg_sh.at[dsti.at[s4]], sem_s.at[s2]).wait()

    issue_idx(w, 0, 0)
    issue_idx(w + NW, 1, 1)
    wait_idx(0)
    issue_main(w, 0)

    def edge_block(b, _):
        i2 = b % 2
        i2n = (b + 1) % 2
        i4 = b % 4

        wait_main(i2)

        @pl.when(b >= 1)
        def _():
            wait_scatter(i2n, (b + 3) % 4)

        @pl.when(b < BPW - 2)
        def _():
            issue_idx(w + (b + 2) * NW, i2, (b + 2) % 4)

        @pl.when(b < BPW - 1)
        def _():
            wait_idx(i2n)
            issue_main(w + (b + 1) * NW, i2n)

        @plsc.parallel_loop(0, EB, step=1, unroll=4)
        def _(r):
            for j in range(D // LANES):
                sl = pl.ds(j * LANES, LANES)
                rows_v[i2, r, sl] = jnp.maximum(rows_v[i2, r, sl] + ew_v[i2, r, sl], 0.0)

        pltpu.async_copy(rows_v.at[i2], agg_sh.at[dsti.at[i4]], sem_s.at[i2], add=True)
        return 0
    lax.fori_loop(0, BPW, edge_block, 0)

    wait_scatter((BPW - 1) % 2, (BPW - 1) % 4)
    plsc.subcore_barrier()

    for k in range(5):
        r0 = (sid * 5 + k) * 128
        pltpu.sync_copy(agg_sh.at[pl.ds(r0, 128)], out_hbm.at[cid, pl.ds(r0, 128)])


_sc_edge = pl.kernel(
    _sc_edge_body,
    out_type=jax.ShapeDtypeStruct((2, NPAD, D), jnp.float32),
    mesh=plsc.VectorSubcoreMesh(core_axis_name="c", subcore_axis_name="s",
                                num_cores=2, num_subcores=16),
    scratch_types=[
        pltpu.VMEM((2, EB, D), jnp.float32),
        pltpu.VMEM((2, EB, D), jnp.float32),
        pltpu.VMEM((2, EB), jnp.int32),
        pltpu.VMEM((4, EB), jnp.int32),
        pltpu.VMEM_SHARED((NPAD, D), jnp.float32),
        pltpu.SemaphoreType.DMA((2,)),
        pltpu.SemaphoreType.DMA((2,)),
        pltpu.SemaphoreType.DMA((2,)),
        pltpu.SemaphoreType.DMA((2,)),
        pltpu.SemaphoreType.DMA((2,)),
    ],
)


def _tc_dense_body(x_ref, agg_ref, seg_ref, W1_ref, b1_ref, W2_ref, b2_ref,
                   eps_ref, Wl_ref, bl_ref, out_ref, h_ref, pooled_ref):
    h0 = (1.0 + eps_ref[...]) * x_ref[...] + agg_ref[0, :N, :] + agg_ref[1, :N, :]
    t = jnp.maximum(
        jnp.dot(h0, W1_ref[...], preferred_element_type=jnp.float32) + b1_ref[...],
        0.0)
    h_ref[...] = jnp.dot(t, W2_ref[...], preferred_element_type=jnp.float32) + b2_ref[...]

    def pool_g(g, _):
        s0 = seg_ref[g]
        s1 = seg_ref[g + 1]
        c0 = s0 // 8
        n8 = (s1 + 7) // 8 - c0

        def step(j, acc):
            off = (c0 + j) * 8
            blockv = h_ref[pl.ds(off, 8), :]
            rowid = off + lax.broadcasted_iota(jnp.int32, (8, HID), 0)
            m = (rowid >= s0) & (rowid < s1)
            return jnp.maximum(acc, jnp.where(m, blockv, -3.4e38))
        acc = lax.fori_loop(0, n8, step, jnp.full((8, HID), -3.4e38, jnp.float32))
        pooled_ref[pl.ds(g, 1), :] = jnp.max(acc, axis=0, keepdims=True)
        return 0
    lax.fori_loop(0, G, pool_g, 0)

    out_ref[...] = (jnp.dot(pooled_ref[...], Wl_ref[...],
                            preferred_element_type=jnp.float32)
                    + bl_ref[...])


_tc_dense = pl.pallas_call(
    _tc_dense_body,
    out_shape=jax.ShapeDtypeStruct((G, OUT), jnp.float32),
    in_specs=[pl.BlockSpec(memory_space=pltpu.VMEM)] * 2
             + [pl.BlockSpec(memory_space=pltpu.SMEM)]
             + [pl.BlockSpec(memory_space=pltpu.VMEM)] * 7,
    scratch_shapes=[pltpu.VMEM((N, HID), jnp.float32),
                    pltpu.VMEM((G, HID), jnp.float32)],
)


def kernel(x, edge_index, edge_weights, batch, W1, b1, W2, b2, eps, Wl, bl):
    src = edge_index[0]
    dst = edge_index[1]
    pad = EPAD - E
    src_p = jnp.concatenate([src, jnp.zeros((pad,), jnp.int32)])
    dst_p = jnp.concatenate([dst, jnp.full((pad,), NPAD - 1, jnp.int32)])
    seg = jnp.searchsorted(batch, jnp.arange(G + 1, dtype=jnp.int32)).astype(jnp.int32)
    agg = _sc_edge(x, src_p, dst_p, edge_weights)
    out = _tc_dense(x, agg, seg,
                    W1, b1.reshape(1, -1), W2, b2.reshape(1, -1),
                    eps.reshape(1, 1), Wl, bl.reshape(1, -1))
    return out


# speedup vs baseline: 5.9387x; 1.1812x over previous
import jax
import jax.numpy as jnp
from jax import lax
from jax.experimental import pallas as pl
from jax.experimental.pallas import tpu as pltpu
from jax.experimental.pallas import tpu_sc as plsc

N, E, D, HID, OUT, G = 10000, 320000, 128, 128, 128, 64
NPAD = 10240
EB = 64
NB = E // EB
NW = 32
BPW = 158
EPAD = NW * BPW * EB
LANES = 16


def _sc_edge_body(x_hbm, src_hbm, dst_hbm, ew_hbm, out_hbm,
                  rows_v, ew_v, m_v, srci, dsti, agg_sh,
                  sem_is, sem_id, sem_g, sem_e, sem_s):
    cid = lax.axis_index("c")
    sid = lax.axis_index("s")
    w = sid * 2 + cid

    def zrow(i, _):
        for j in range(D // LANES):
            m_v[0, i, pl.ds(j * LANES, LANES)] = jnp.zeros((LANES,), jnp.float32)
        return 0
    lax.fori_loop(0, EB, zrow, 0)
    for k in range(10):
        r0 = (sid * 10 + k) * EB
        pltpu.sync_copy(m_v.at[0], agg_sh.at[pl.ds(r0, EB)])
    plsc.subcore_barrier()

    def issue_idx(blk, s2, s4):
        base = pl.multiple_of(blk * EB, EB)
        pltpu.async_copy(src_hbm.at[pl.ds(base, EB)], srci.at[s2], sem_is.at[s2])
        pltpu.async_copy(dst_hbm.at[pl.ds(base, EB)], dsti.at[s4], sem_id.at[s2])

    def wait_idx(s2):
        pltpu.make_async_copy(src_hbm.at[pl.ds(0, EB)], srci.at[s2], sem_is.at[s2]).wait()
        pltpu.make_async_copy(dst_hbm.at[pl.ds(0, EB)], dsti.at[0], sem_id.at[s2]).wait()

    def issue_main(blk, s2):
        pltpu.async_copy(x_hbm.at[srci.at[s2]], rows_v.at[s2], sem_g.at[s2])
        base_ew = pl.multiple_of(jnp.minimum(blk, NB - 1) * EB, EB)
        pltpu.async_copy(ew_hbm.at[pl.ds(base_ew, EB)], ew_v.at[s2], sem_e.at[s2])

    def wait_main(s2):
        pltpu.make_async_copy(x_hbm.at[srci.at[s2]], rows_v.at[s2], sem_g.at[s2]).wait()
        pltpu.make_async_copy(ew_hbm.at[pl.ds(0, EB)], ew_v.at[s2], sem_e.at[s2]).wait()

    def wait_scatter(s2, s4):
        pltpu.make_async_copy(m_v.at[s2], agg_sh.at[dsti.at[s4]], sem_s.at[s2]).wait()

    issue_idx(w, 0, 0)
    issue_idx(w + NW, 1, 1)
    wait_idx(0)
    issue_main(w, 0)

    def edge_block(b, _):
        i2 = b % 2
        i2n = (b + 1) % 2
        i4 = b % 4

        @pl.when(b >= 2)
        def _():
            wait_scatter(i2, (b + 2) % 4)

        wait_main(i2)

        @pl.when(b < BPW - 2)
        def _():
            issue_idx(w + (b + 2) * NW, i2, (b + 2) % 4)

        @pl.when(b < BPW - 1)
        def _():
            wait_idx(i2n)
            issue_main(w + (b + 1) * NW, i2n)

        @plsc.parallel_loop(0, EB, step=1, unroll=4)
        def _(r):
            for j in range(D // 32):
                x32 = plsc.bitcast(rows_v[i2, r, pl.ds(j * LANES, LANES)],
                                   jnp.bfloat16)
                xa, xb = plsc.unpack(x32, format=plsc.PackFormat.INTERLEAVED)
                sla = pl.ds(j * 32, LANES)
                slb = pl.ds(j * 32 + LANES, LANES)
                m_v[i2, r, sla] = jnp.maximum(xa + ew_v[i2, r, sla], 0.0)
                m_v[i2, r, slb] = jnp.maximum(xb + ew_v[i2, r, slb], 0.0)

        pltpu.async_copy(m_v.at[i2], agg_sh.at[dsti.at[i4]], sem_s.at[i2], add=True)
        return 0
    lax.fori_loop(0, BPW, edge_block, 0)

    wait_scatter((BPW - 2) % 2, (BPW - 2) % 4)
    wait_scatter((BPW - 1) % 2, (BPW - 1) % 4)
    plsc.subcore_barrier()

    for k in range(5):
        r0 = (sid * 5 + k) * 128
        pltpu.sync_copy(agg_sh.at[pl.ds(r0, 128)], out_hbm.at[cid, pl.ds(r0, 128)])


_sc_edge = pl.kernel(
    _sc_edge_body,
    out_type=jax.ShapeDtypeStruct((2, NPAD, D), jnp.float32),
    mesh=plsc.VectorSubcoreMesh(core_axis_name="c", subcore_axis_name="s",
                                num_cores=2, num_subcores=16),
    compiler_params=pltpu.CompilerParams(use_tc_tiling_on_sc=False,
                                         needs_layout_passes=False),
    scratch_types=[
        pltpu.VMEM((2, EB, D // 2), jnp.float32),
        pltpu.VMEM((2, EB, D), jnp.float32),
        pltpu.VMEM((2, EB, D), jnp.float32),
        pltpu.VMEM((2, EB), jnp.int32),
        pltpu.VMEM((4, EB), jnp.int32),
        pltpu.VMEM_SHARED((NPAD, D), jnp.float32),
        pltpu.SemaphoreType.DMA((2,)),
        pltpu.SemaphoreType.DMA((2,)),
        pltpu.SemaphoreType.DMA((2,)),
        pltpu.SemaphoreType.DMA((2,)),
        pltpu.SemaphoreType.DMA((2,)),
    ],
)


def _tc_dense_body(x_ref, agg_ref, seg_ref, W1_ref, b1_ref, W2_ref, b2_ref,
                   eps_ref, Wl_ref, bl_ref, out_ref, h_ref, pooled_ref):
    h0 = (1.0 + eps_ref[...]) * x_ref[...] + agg_ref[0, :N, :] + agg_ref[1, :N, :]
    t = jnp.maximum(
        jnp.dot(h0, W1_ref[...], preferred_element_type=jnp.float32) + b1_ref[...],
        0.0)
    h_ref[...] = jnp.dot(t, W2_ref[...], preferred_element_type=jnp.float32) + b2_ref[...]

    def pool_g(g, _):
        s0 = seg_ref[g]
        s1 = seg_ref[g + 1]
        c0 = s0 // 8
        n8 = (s1 + 7) // 8 - c0

        def step(j, acc):
            off = (c0 + j) * 8
            blockv = h_ref[pl.ds(off, 8), :]
            rowid = off + lax.broadcasted_iota(jnp.int32, (8, HID), 0)
            m = (rowid >= s0) & (rowid < s1)
            return jnp.maximum(acc, jnp.where(m, blockv, -3.4e38))
        acc = lax.fori_loop(0, n8, step, jnp.full((8, HID), -3.4e38, jnp.float32))
        pooled_ref[pl.ds(g, 1), :] = jnp.max(acc, axis=0, keepdims=True)
        return 0
    lax.fori_loop(0, G, pool_g, 0)

    out_ref[...] = (jnp.dot(pooled_ref[...], Wl_ref[...],
                            preferred_element_type=jnp.float32)
                    + bl_ref[...])


_tc_dense = pl.pallas_call(
    _tc_dense_body,
    out_shape=jax.ShapeDtypeStruct((G, OUT), jnp.float32),
    in_specs=[pl.BlockSpec(memory_space=pltpu.VMEM)] * 2
             + [pl.BlockSpec(memory_space=pltpu.SMEM)]
             + [pl.BlockSpec(memory_space=pltpu.VMEM)] * 7,
    scratch_shapes=[pltpu.VMEM((N, HID), jnp.float32),
                    pltpu.VMEM((G, HID), jnp.float32)],
)


_QPERM = tuple(g * 32 + (t // 2 if t % 2 == 0 else 16 + t // 2)
               for g in range(D // 32) for t in range(32))


def kernel(x, edge_index, edge_weights, batch, W1, b1, W2, b2, eps, Wl, bl):
    src = edge_index[0]
    dst = edge_index[1]
    pad = EPAD - E
    src_p = jnp.concatenate([src, jnp.zeros((pad,), jnp.int32)])
    dst_p = jnp.concatenate([dst, jnp.full((pad,), NPAD - 1, jnp.int32)])
    seg = jnp.searchsorted(batch, jnp.arange(G + 1, dtype=jnp.int32)).astype(jnp.int32)
    xq = x[:, jnp.array(_QPERM, dtype=jnp.int32)].astype(jnp.bfloat16)
    xq32 = lax.bitcast_convert_type(xq.reshape(N, D // 2, 2), jnp.float32)
    agg = _sc_edge(xq32, src_p, dst_p, edge_weights)
    out = _tc_dense(x, agg, seg,
                    W1, b1.reshape(1, -1), W2, b2.reshape(1, -1),
                    eps.reshape(1, 1), Wl, bl.reshape(1, -1))
    return out


# speedup vs baseline: 6.3616x; 1.0712x over previous
import jax
import jax.numpy as jnp
from jax import lax
from jax.experimental import pallas as pl
from jax.experimental.pallas import tpu as pltpu
from jax.experimental.pallas import tpu_sc as plsc

N, E, D, HID, OUT, G = 10000, 320000, 128, 128, 128, 64
NPAD = 10240
EB = 64
NB = E // EB
NW = 32
BPW = 158
EPAD = NW * BPW * EB
LANES = 16


def _sc_edge_body(x_hbm, src_hbm, dst_hbm, ew_hbm, out_hbm,
                  rows_v, ew_v, m_v, srci, dsti, agg_sh,
                  sem_is, sem_id, sem_g, sem_e, sem_s):
    cid = lax.axis_index("c")
    sid = lax.axis_index("s")
    w = sid * 2 + cid

    def zrow(i, _):
        for j in range(D // LANES):
            m_v[0, i, pl.ds(j * LANES, LANES)] = jnp.zeros((LANES,), jnp.float32)
        return 0
    lax.fori_loop(0, EB, zrow, 0)
    for k in range(10):
        r0 = (sid * 10 + k) * EB
        pltpu.sync_copy(m_v.at[0], agg_sh.at[pl.ds(r0, EB)])
    plsc.subcore_barrier()

    def issue_idx(blk, s2, s4):
        base = pl.multiple_of(blk * EB, EB)
        pltpu.async_copy(src_hbm.at[pl.ds(base, EB)], srci.at[s2], sem_is.at[s2])
        pltpu.async_copy(dst_hbm.at[pl.ds(base, EB)], dsti.at[s4], sem_id.at[s2])

    def wait_idx(s2):
        pltpu.make_async_copy(src_hbm.at[pl.ds(0, EB)], srci.at[s2], sem_is.at[s2]).wait()
        pltpu.make_async_copy(dst_hbm.at[pl.ds(0, EB)], dsti.at[0], sem_id.at[s2]).wait()

    def issue_main(blk, s2):
        pltpu.async_copy(x_hbm.at[srci.at[s2]], rows_v.at[s2], sem_g.at[s2])
        base_ew = pl.multiple_of(jnp.minimum(blk, NB - 1) * EB, EB)
        pltpu.async_copy(ew_hbm.at[pl.ds(base_ew, EB)], ew_v.at[s2], sem_e.at[s2])

    def wait_main(s2):
        pltpu.make_async_copy(x_hbm.at[srci.at[s2]], rows_v.at[s2], sem_g.at[s2]).wait()
        pltpu.make_async_copy(ew_hbm.at[pl.ds(0, EB)], ew_v.at[s2], sem_e.at[s2]).wait()

    def wait_scatter(s2, s4):
        pltpu.make_async_copy(m_v.at[s2], agg_sh.at[dsti.at[s4]], sem_s.at[s2]).wait()

    issue_idx(w, 0, 0)
    issue_idx(w + NW, 1, 1)
    wait_idx(0)
    issue_main(w, 0)

    def edge_block(b, _):
        i2 = b % 2
        i2n = (b + 1) % 2
        i4 = b % 4

        @pl.when(b >= 2)
        def _():
            wait_scatter(i2, (b + 2) % 4)

        wait_main(i2)

        @pl.when(b < BPW - 2)
        def _():
            issue_idx(w + (b + 2) * NW, i2, (b + 2) % 4)

        @pl.when(b < BPW - 1)
        def _():
            wait_idx(i2n)
            issue_main(w + (b + 1) * NW, i2n)

        @plsc.parallel_loop(0, EB, step=1, unroll=8)
        def _(r):
            for j in range(D // 32):
                x32 = plsc.bitcast(rows_v[i2, r, pl.ds(j * LANES, LANES)],
                                   jnp.bfloat16)
                xa, xb = plsc.unpack(x32, format=plsc.PackFormat.INTERLEAVED)
                sla = pl.ds(j * 32, LANES)
                slb = pl.ds(j * 32 + LANES, LANES)
                m_v[i2, r, sla] = jnp.maximum(xa + ew_v[i2, r, sla], 0.0)
                m_v[i2, r, slb] = jnp.maximum(xb + ew_v[i2, r, slb], 0.0)

        pltpu.async_copy(m_v.at[i2], agg_sh.at[dsti.at[i4]], sem_s.at[i2], add=True)
        return 0
    lax.fori_loop(0, BPW, edge_block, 0)

    wait_scatter((BPW - 2) % 2, (BPW - 2) % 4)
    wait_scatter((BPW - 1) % 2, (BPW - 1) % 4)
    plsc.subcore_barrier()

    for k in range(5):
        r0 = (sid * 5 + k) * 128
        pltpu.sync_copy(agg_sh.at[pl.ds(r0, 128)], out_hbm.at[cid, pl.ds(r0, 128)])


_sc_edge = pl.kernel(
    _sc_edge_body,
    out_type=jax.ShapeDtypeStruct((2, NPAD, D), jnp.float32),
    mesh=plsc.VectorSubcoreMesh(core_axis_name="c", subcore_axis_name="s",
                                num_cores=2, num_subcores=16),
    compiler_params=pltpu.CompilerParams(use_tc_tiling_on_sc=False,
                                         needs_layout_passes=False),
    scratch_types=[
        pltpu.VMEM((2, EB, D // 2), jnp.float32),
        pltpu.VMEM((2, EB, D), jnp.float32),
        pltpu.VMEM((2, EB, D), jnp.float32),
        pltpu.VMEM((2, EB), jnp.int32),
        pltpu.VMEM((4, EB), jnp.int32),
        pltpu.VMEM_SHARED((NPAD, D), jnp.float32),
        pltpu.SemaphoreType.DMA((2,)),
        pltpu.SemaphoreType.DMA((2,)),
        pltpu.SemaphoreType.DMA((2,)),
        pltpu.SemaphoreType.DMA((2,)),
        pltpu.SemaphoreType.DMA((2,)),
    ],
)


def _tc_dense_body(x_ref, agg_ref, seg_ref, W1_ref, b1_ref, W2_ref, b2_ref,
                   eps_ref, Wl_ref, bl_ref, out_ref, h_ref, pooled_ref):
    h0 = (1.0 + eps_ref[...]) * x_ref[...] + agg_ref[0, :N, :] + agg_ref[1, :N, :]
    t = jnp.maximum(
        jnp.dot(h0, W1_ref[...], preferred_element_type=jnp.float32) + b1_ref[...],
        0.0)
    h_ref[...] = jnp.dot(t, W2_ref[...], preferred_element_type=jnp.float32) + b2_ref[...]

    def pool_g(g, _):
        s0 = seg_ref[g]
        s1 = seg_ref[g + 1]
        c0 = s0 // 8
        n8 = (s1 + 7) // 8 - c0

        def step(j, acc):
            off = (c0 + j) * 8
            blockv = h_ref[pl.ds(off, 8), :]
            rowid = off + lax.broadcasted_iota(jnp.int32, (8, HID), 0)
            m = (rowid >= s0) & (rowid < s1)
            return jnp.maximum(acc, jnp.where(m, blockv, -3.4e38))
        acc = lax.fori_loop(0, n8, step, jnp.full((8, HID), -3.4e38, jnp.float32))
        pooled_ref[pl.ds(g, 1), :] = jnp.max(acc, axis=0, keepdims=True)
        return 0
    lax.fori_loop(0, G, pool_g, 0)

    out_ref[...] = (jnp.dot(pooled_ref[...], Wl_ref[...],
                            preferred_element_type=jnp.float32)
                    + bl_ref[...])


_tc_dense = pl.pallas_call(
    _tc_dense_body,
    out_shape=jax.ShapeDtypeStruct((G, OUT), jnp.float32),
    in_specs=[pl.BlockSpec(memory_space=pltpu.VMEM)] * 2
             + [pl.BlockSpec(memory_space=pltpu.SMEM)]
             + [pl.BlockSpec(memory_space=pltpu.VMEM)] * 7,
    scratch_shapes=[pltpu.VMEM((N, HID), jnp.float32),
                    pltpu.VMEM((G, HID), jnp.float32)],
)


def kernel(x, edge_index, edge_weights, batch, W1, b1, W2, b2, eps, Wl, bl):
    src = edge_index[0]
    dst = edge_index[1]
    pad = EPAD - E
    src_p = jnp.concatenate([src, jnp.zeros((pad,), jnp.int32)])
    dst_p = jnp.concatenate([dst, jnp.full((pad,), NPAD - 1, jnp.int32)])
    seg = jnp.searchsorted(batch, jnp.arange(G + 1, dtype=jnp.int32)).astype(jnp.int32)
    xq = (x.reshape(N, D // 32, 2, 16).swapaxes(2, 3)
          .reshape(N, D).astype(jnp.bfloat16))
    xq32 = lax.bitcast_convert_type(xq.reshape(N, D // 2, 2), jnp.float32)
    agg = _sc_edge(xq32, src_p, dst_p, edge_weights)
    out = _tc_dense(x, agg, seg,
                    W1, b1.reshape(1, -1), W2, b2.reshape(1, -1),
                    eps.reshape(1, 1), Wl, bl.reshape(1, -1))
    return out


# speedup vs baseline: 9.1667x; 1.4410x over previous
import jax
import jax.numpy as jnp
from jax import lax
from jax.experimental import pallas as pl
from jax.experimental.pallas import tpu as pltpu
from jax.experimental.pallas import tpu_sc as plsc

N, E, D, HID, OUT, G = 10000, 320000, 128, 128, 128, 64
NPAD = 10240
EB = 80
NB = E // EB
NW = 32
BPW = NB // NW
LANES = 16


def _sc_edge_body(x_hbm, src_hbm, dst_hbm, ew_hbm, out_hbm,
                  rows_v, m_v, srci, dsti, agg_sh,
                  sem_is, sem_id, sem_g, sem_e, sem_s):
    cid = lax.axis_index("c")
    sid = lax.axis_index("s")
    w = sid * 2 + cid

    def zrow(i, _):
        for j in range(D // LANES):
            m_v[0, i, pl.ds(j * LANES, LANES)] = jnp.zeros((LANES,), jnp.float32)
        return 0
    lax.fori_loop(0, EB, zrow, 0)
    for k in range(8):
        r0 = (sid * 8 + k) * EB
        pltpu.sync_copy(m_v.at[0], agg_sh.at[pl.ds(r0, EB)])
    plsc.subcore_barrier()

    def issue_idx(blk, s2, s4):
        base = pl.multiple_of(blk * EB, 8)
        pltpu.async_copy(src_hbm.at[pl.ds(base, EB)], srci.at[s2], sem_is.at[s2])
        pltpu.async_copy(dst_hbm.at[pl.ds(base, EB)], dsti.at[s4], sem_id.at[s2])

    def wait_idx(s2):
        pltpu.make_async_copy(src_hbm.at[pl.ds(0, EB)], srci.at[s2], sem_is.at[s2]).wait()
        pltpu.make_async_copy(dst_hbm.at[pl.ds(0, EB)], dsti.at[0], sem_id.at[s2]).wait()

    def issue_ew(blk, s3):
        base = pl.multiple_of(blk * EB, 8)
        pltpu.async_copy(ew_hbm.at[pl.ds(base, EB)], m_v.at[s3], sem_e.at[s3])

    def wait_ew(s3):
        pltpu.make_async_copy(ew_hbm.at[pl.ds(0, EB)], m_v.at[s3], sem_e.at[s3]).wait()

    def issue_gather(s2):
        pltpu.async_copy(x_hbm.at[srci.at[s2]], rows_v.at[s2], sem_g.at[s2])

    def wait_gather(s2):
        pltpu.make_async_copy(x_hbm.at[srci.at[s2]], rows_v.at[s2], sem_g.at[s2]).wait()

    def wait_scatter(s3, s4):
        pltpu.make_async_copy(m_v.at[s3], agg_sh.at[dsti.at[s4]], sem_s.at[s3]).wait()

    issue_idx(w, 0, 0)
    issue_idx(w + NW, 1, 1)
    issue_ew(w, 0)
    wait_idx(0)
    issue_gather(0)

    def edge_block(b, _):
        i2 = b % 2
        i2n = (b + 1) % 2
        e3 = b % 3
        e3n = (b + 1) % 3

        @pl.when(b >= 2)
        def _():
            wait_scatter((b - 2) % 3, (b - 2) % 4)

        @pl.when(b < BPW - 1)
        def _():
            issue_ew(w + (b + 1) * NW, e3n)

        wait_ew(e3)
        wait_gather(i2)

        @pl.when(b < BPW - 2)
        def _():
            issue_idx(w + (b + 2) * NW, i2, (b + 2) % 4)

        @pl.when(b < BPW - 1)
        def _():
            wait_idx(i2n)
            issue_gather(i2n)

        @plsc.parallel_loop(0, EB, step=1, unroll=8)
        def _(r):
            for j in range(D // 32):
                x32 = plsc.bitcast(rows_v[i2, r, pl.ds(j * LANES, LANES)],
                                   jnp.bfloat16)
                xa, xb = plsc.unpack(x32, format=plsc.PackFormat.INTERLEAVED)
                sla = pl.ds(j * 32, LANES)
                slb = pl.ds(j * 32 + LANES, LANES)
                m_v[e3, r, sla] = jnp.maximum(xa + m_v[e3, r, sla], 0.0)
                m_v[e3, r, slb] = jnp.maximum(xb + m_v[e3, r, slb], 0.0)

        pltpu.async_copy(m_v.at[e3], agg_sh.at[dsti.at[b % 4]], sem_s.at[e3], add=True)
        return 0
    lax.fori_loop(0, BPW, edge_block, 0)

    wait_scatter((BPW - 2) % 3, (BPW - 2) % 4)
    wait_scatter((BPW - 1) % 3, (BPW - 1) % 4)
    plsc.subcore_barrier()

    for k in range(5):
        r0 = (sid * 5 + k) * 128
        pltpu.sync_copy(agg_sh.at[pl.ds(r0, 128)], out_hbm.at[cid, pl.ds(r0, 128)])


_sc_edge = pl.kernel(
    _sc_edge_body,
    out_type=jax.ShapeDtypeStruct((2, NPAD, D), jnp.float32),
    mesh=plsc.VectorSubcoreMesh(core_axis_name="c", subcore_axis_name="s",
                                num_cores=2, num_subcores=16),
    compiler_params=pltpu.CompilerParams(use_tc_tiling_on_sc=False,
                                         needs_layout_passes=False),
    scratch_types=[
        pltpu.VMEM((2, EB, D // 2), jnp.float32),
        pltpu.VMEM((3, EB, D), jnp.float32),
        pltpu.VMEM((2, EB), jnp.int32),
        pltpu.VMEM((4, EB), jnp.int32),
        pltpu.VMEM_SHARED((NPAD, D), jnp.float32),
        pltpu.SemaphoreType.DMA((2,)),
        pltpu.SemaphoreType.DMA((2,)),
        pltpu.SemaphoreType.DMA((2,)),
        pltpu.SemaphoreType.DMA((3,)),
        pltpu.SemaphoreType.DMA((3,)),
    ],
)


def _tc_dense_body(x_ref, agg_ref, seg_ref, W1_ref, b1_ref, W2_ref, b2_ref,
                   eps_ref, Wl_ref, bl_ref, out_ref, h_ref, pooled_ref):
    h0 = (1.0 + eps_ref[...]) * x_ref[...] + agg_ref[0, :N, :] + agg_ref[1, :N, :]
    t = jnp.maximum(
        jnp.dot(h0, W1_ref[...], preferred_element_type=jnp.float32) + b1_ref[...],
        0.0)
    h_ref[...] = jnp.dot(t, W2_ref[...], preferred_element_type=jnp.float32) + b2_ref[...]

    def pool_g(g, _):
        s0 = seg_ref[g]
        s1 = seg_ref[g + 1]
        c0 = s0 // 8
        n8 = (s1 + 7) // 8 - c0

        def step(j, acc):
            off = (c0 + j) * 8
            blockv = h_ref[pl.ds(off, 8), :]
            rowid = off + lax.broadcasted_iota(jnp.int32, (8, HID), 0)
            m = (rowid >= s0) & (rowid < s1)
            return jnp.maximum(acc, jnp.where(m, blockv, -3.4e38))
        acc = lax.fori_loop(0, n8, step, jnp.full((8, HID), -3.4e38, jnp.float32))
        pooled_ref[pl.ds(g, 1), :] = jnp.max(acc, axis=0, keepdims=True)
        return 0
    lax.fori_loop(0, G, pool_g, 0)

    out_ref[...] = (jnp.dot(pooled_ref[...], Wl_ref[...],
                            preferred_element_type=jnp.float32)
                    + bl_ref[...])


_tc_dense = pl.pallas_call(
    _tc_dense_body,
    out_shape=jax.ShapeDtypeStruct((G, OUT), jnp.float32),
    in_specs=[pl.BlockSpec(memory_space=pltpu.VMEM)] * 2
             + [pl.BlockSpec(memory_space=pltpu.SMEM)]
             + [pl.BlockSpec(memory_space=pltpu.VMEM)] * 7,
    scratch_shapes=[pltpu.VMEM((N, HID), jnp.float32),
                    pltpu.VMEM((G, HID), jnp.float32)],
)


def kernel(x, edge_index, edge_weights, batch, W1, b1, W2, b2, eps, Wl, bl):
    src = edge_index[0]
    dst = edge_index[1]
    seg = jnp.searchsorted(batch, jnp.arange(G + 1, dtype=jnp.int32)).astype(jnp.int32)
    xq = (x.reshape(N, D // 32, 2, 16).swapaxes(2, 3)
          .reshape(N, D).astype(jnp.bfloat16))
    xq32 = lax.bitcast_convert_type(xq.reshape(N, D // 2, 2), jnp.float32)
    agg = _sc_edge(xq32, src, dst, edge_weights)
    out = _tc_dense(x, agg, seg,
                    W1, b1.reshape(1, -1), W2, b2.reshape(1, -1),
                    eps.reshape(1, 1), Wl, bl.reshape(1, -1))
    return out


# speedup vs baseline: 9.1695x; 1.0003x over previous
import jax
import jax.numpy as jnp
from jax import lax
from jax.experimental import pallas as pl
from jax.experimental.pallas import tpu as pltpu
from jax.experimental.pallas import tpu_sc as plsc

N, E, D, HID, OUT, G = 10000, 320000, 128, 128, 128, 64
NPAD = 10240
EB = 80
NB = E // EB
NW = 32
BPW = NB // NW
LANES = 16


def _sc_edge_body(x_hbm, src_hbm, dst_hbm, ew_hbm, out_hbm,
                  rows_v, m_v, srci, dsti, agg_sh,
                  sem_is, sem_id, sem_g, sem_e, sem_s):
    cid = lax.axis_index("c")
    sid = lax.axis_index("s")
    w = sid * 2 + cid

    def zrow(i, _):
        for j in range(D // LANES):
            m_v[0, i, pl.ds(j * LANES, LANES)] = jnp.zeros((LANES,), jnp.float32)
        return 0
    lax.fori_loop(0, EB, zrow, 0)
    for k in range(8):
        r0 = (sid * 8 + k) * EB
        pltpu.sync_copy(m_v.at[0], agg_sh.at[pl.ds(r0, EB)])
    plsc.subcore_barrier()

    def issue_idx(blk, s2, s4):
        base = pl.multiple_of(blk * EB, 8)
        pltpu.async_copy(src_hbm.at[pl.ds(base, EB)], srci.at[s2], sem_is.at[s2])
        pltpu.async_copy(dst_hbm.at[pl.ds(base, EB)], dsti.at[s4], sem_id.at[s2])

    def wait_idx(s2):
        pltpu.make_async_copy(src_hbm.at[pl.ds(0, EB)], srci.at[s2], sem_is.at[s2]).wait()
        pltpu.make_async_copy(dst_hbm.at[pl.ds(0, EB)], dsti.at[0], sem_id.at[s2]).wait()

    def issue_ew(blk, s3):
        base = pl.multiple_of(blk * EB, 8)
        pltpu.async_copy(ew_hbm.at[pl.ds(base, EB)], m_v.at[s3], sem_e.at[s3])

    def wait_ew(s3):
        pltpu.make_async_copy(ew_hbm.at[pl.ds(0, EB)], m_v.at[s3], sem_e.at[s3]).wait()

    def issue_gather(s2):
        pltpu.async_copy(x_hbm.at[srci.at[s2]], rows_v.at[s2], sem_g.at[s2])

    def wait_gather(s2):
        pltpu.make_async_copy(x_hbm.at[srci.at[s2]], rows_v.at[s2], sem_g.at[s2]).wait()

    def wait_scatter(s3, s4):
        pltpu.make_async_copy(m_v.at[s3], agg_sh.at[dsti.at[s4]], sem_s.at[s3]).wait()

    issue_idx(w, 0, 0)
    issue_idx(w + NW, 1, 1)
    issue_ew(w, 0)
    wait_idx(0)
    issue_gather(0)

    def edge_block(b, _):
        i2 = b % 2
        i2n = (b + 1) % 2
        e3 = b % 3
        e3n = (b + 1) % 3

        @pl.when(b >= 2)
        def _():
            wait_scatter((b - 2) % 3, (b - 2) % 4)

        @pl.when(b < BPW - 1)
        def _():
            issue_ew(w + (b + 1) * NW, e3n)

        wait_ew(e3)
        wait_gather(i2)

        @pl.when(b < BPW - 2)
        def _():
            issue_idx(w + (b + 2) * NW, i2, (b + 2) % 4)

        @pl.when(b < BPW - 1)
        def _():
            wait_idx(i2n)
            issue_gather(i2n)

        @plsc.parallel_loop(0, EB, step=1, unroll=8)
        def _(r):
            for j in range(D // 32):
                x32 = plsc.bitcast(rows_v[i2, r, pl.ds(j * LANES, LANES)],
                                   jnp.bfloat16)
                xa, xb = plsc.unpack(x32, format=plsc.PackFormat.INTERLEAVED)
                sla = pl.ds(j * 32, LANES)
                slb = pl.ds(j * 32 + LANES, LANES)
                m_v[e3, r, sla] = jnp.maximum(xa + m_v[e3, r, sla], 0.0)
                m_v[e3, r, slb] = jnp.maximum(xb + m_v[e3, r, slb], 0.0)

        pltpu.async_copy(m_v.at[e3], agg_sh.at[dsti.at[b % 4]], sem_s.at[e3], add=True)
        return 0
    lax.fori_loop(0, BPW, edge_block, 0)

    wait_scatter((BPW - 2) % 3, (BPW - 2) % 4)
    wait_scatter((BPW - 1) % 3, (BPW - 1) % 4)
    plsc.subcore_barrier()

    for k in range(5):
        r0 = (sid * 5 + k) * 128
        pltpu.sync_copy(agg_sh.at[pl.ds(r0, 128)], out_hbm.at[cid, pl.ds(r0, 128)])


_sc_edge = pl.kernel(
    _sc_edge_body,
    out_type=jax.ShapeDtypeStruct((2, NPAD, D), jnp.float32),
    mesh=plsc.VectorSubcoreMesh(core_axis_name="c", subcore_axis_name="s",
                                num_cores=2, num_subcores=16),
    compiler_params=pltpu.CompilerParams(use_tc_tiling_on_sc=False,
                                         needs_layout_passes=False),
    scratch_types=[
        pltpu.VMEM((2, EB, D // 2), jnp.float32),
        pltpu.VMEM((3, EB, D), jnp.float32),
        pltpu.VMEM((2, EB), jnp.int32),
        pltpu.VMEM((4, EB), jnp.int32),
        pltpu.VMEM_SHARED((NPAD, D), jnp.float32),
        pltpu.SemaphoreType.DMA((2,)),
        pltpu.SemaphoreType.DMA((2,)),
        pltpu.SemaphoreType.DMA((2,)),
        pltpu.SemaphoreType.DMA((3,)),
        pltpu.SemaphoreType.DMA((3,)),
    ],
)


def _tc_dense_body(x_ref, agg_ref, seg_ref, W1_ref, b1_ref, W2_ref, b2_ref,
                   eps_ref, Wl_ref, bl_ref, out_ref, h_ref, pooled_ref):
    h0 = (1.0 + eps_ref[...]) * x_ref[...] + agg_ref[0, :N, :] + agg_ref[1, :N, :]
    t = jnp.maximum(
        jnp.dot(h0.astype(jnp.bfloat16), W1_ref[...],
                preferred_element_type=jnp.float32) + b1_ref[...],
        0.0)
    h_ref[...] = jnp.dot(t.astype(jnp.bfloat16), W2_ref[...],
                         preferred_element_type=jnp.float32) + b2_ref[...]

    def pool_g(g, _):
        s0 = seg_ref[g]
        s1 = seg_ref[g + 1]
        c0 = s0 // 8
        n8 = (s1 + 7) // 8 - c0

        def step(j, acc):
            off = (c0 + j) * 8
            blockv = h_ref[pl.ds(off, 8), :]
            rowid = off + lax.broadcasted_iota(jnp.int32, (8, HID), 0)
            m = (rowid >= s0) & (rowid < s1)
            return jnp.maximum(acc, jnp.where(m, blockv, -3.4e38))
        acc = lax.fori_loop(0, n8, step, jnp.full((8, HID), -3.4e38, jnp.float32))
        pooled_ref[pl.ds(g, 1), :] = jnp.max(acc, axis=0, keepdims=True)
        return 0
    lax.fori_loop(0, G, pool_g, 0)

    out_ref[...] = (jnp.dot(pooled_ref[...], Wl_ref[...],
                            preferred_element_type=jnp.float32)
                    + bl_ref[...])


_tc_dense = pl.pallas_call(
    _tc_dense_body,
    out_shape=jax.ShapeDtypeStruct((G, OUT), jnp.float32),
    in_specs=[pl.BlockSpec(memory_space=pltpu.VMEM)] * 2
             + [pl.BlockSpec(memory_space=pltpu.SMEM)]
             + [pl.BlockSpec(memory_space=pltpu.VMEM)] * 7,
    scratch_shapes=[pltpu.VMEM((N, HID), jnp.float32),
                    pltpu.VMEM((G, HID), jnp.float32)],
)


def kernel(x, edge_index, edge_weights, batch, W1, b1, W2, b2, eps, Wl, bl):
    src = edge_index[0]
    dst = edge_index[1]
    seg = jnp.searchsorted(batch, jnp.arange(G + 1, dtype=jnp.int32)).astype(jnp.int32)
    xq = (x.reshape(N, D // 32, 2, 16).swapaxes(2, 3)
          .reshape(N, D).astype(jnp.bfloat16))
    xq32 = lax.bitcast_convert_type(xq.reshape(N, D // 2, 2), jnp.float32)
    agg = _sc_edge(xq32, src, dst, edge_weights)
    out = _tc_dense(x, agg, seg,
                    W1.astype(jnp.bfloat16), b1.reshape(1, -1),
                    W2.astype(jnp.bfloat16), b2.reshape(1, -1),
                    eps.reshape(1, 1), Wl, bl.reshape(1, -1))
    return out


# speedup vs baseline: 9.4106x; 1.0263x over previous
import jax
import jax.numpy as jnp
from jax import lax
from jax.experimental import pallas as pl
from jax.experimental.pallas import tpu as pltpu
from jax.experimental.pallas import tpu_sc as plsc

N, E, D, HID, OUT, G = 10000, 320000, 128, 128, 128, 64
NPAD = 10240
EB = 80
NB = E // EB
NW = 32
BPW = NB // NW
LANES = 16


def _sc_edge_body(x_hbm, src_hbm, dst_hbm, ew_hbm, out_hbm,
                  rows_v, m_v, srci, dsti, agg_sh,
                  sem_is, sem_id, sem_g, sem_e, sem_s):
    cid = lax.axis_index("c")
    sid = lax.axis_index("s")
    w = sid * 2 + cid

    def zrow(i, _):
        for j in range(D // LANES):
            m_v[0, i, pl.ds(j * LANES, LANES)] = jnp.zeros((LANES,), jnp.float32)
        return 0
    lax.fori_loop(0, EB, zrow, 0)
    for k in range(8):
        r0 = (sid * 8 + k) * EB
        pltpu.sync_copy(m_v.at[0], agg_sh.at[pl.ds(r0, EB)])
    plsc.subcore_barrier()

    def issue_idx(blk, s2, s4):
        base = pl.multiple_of(blk * EB, 8)
        pltpu.async_copy(src_hbm.at[pl.ds(base, EB)], srci.at[s2], sem_is.at[s2])
        pltpu.async_copy(dst_hbm.at[pl.ds(base, EB)], dsti.at[s4], sem_id.at[s2])

    def wait_idx(s2):
        pltpu.make_async_copy(src_hbm.at[pl.ds(0, EB)], srci.at[s2], sem_is.at[s2]).wait()
        pltpu.make_async_copy(dst_hbm.at[pl.ds(0, EB)], dsti.at[0], sem_id.at[s2]).wait()

    def issue_ew(blk, s3):
        base = pl.multiple_of(blk * EB, 8)
        pltpu.async_copy(ew_hbm.at[pl.ds(base, EB)], m_v.at[s3], sem_e.at[s3])

    def wait_ew(s3):
        pltpu.make_async_copy(ew_hbm.at[pl.ds(0, EB)], m_v.at[s3], sem_e.at[s3]).wait()

    def issue_gather(s2):
        pltpu.async_copy(x_hbm.at[srci.at[s2]], rows_v.at[s2], sem_g.at[s2])

    def wait_gather(s2):
        pltpu.make_async_copy(x_hbm.at[srci.at[s2]], rows_v.at[s2], sem_g.at[s2]).wait()

    def wait_scatter(s3, s4):
        pltpu.make_async_copy(m_v.at[s3], agg_sh.at[dsti.at[s4]], sem_s.at[s3]).wait()

    issue_idx(w, 0, 0)
    issue_idx(w + NW, 1, 1)
    issue_ew(w, 0)
    wait_idx(0)
    issue_gather(0)

    def edge_block(b, _):
        i2 = b % 2
        i2n = (b + 1) % 2
        e3 = b % 3
        e3n = (b + 1) % 3

        @pl.when(b >= 2)
        def _():
            wait_scatter((b - 2) % 3, (b - 2) % 4)

        @pl.when(b < BPW - 1)
        def _():
            issue_ew(w + (b + 1) * NW, e3n)

        wait_ew(e3)
        wait_gather(i2)

        @pl.when(b < BPW - 2)
        def _():
            issue_idx(w + (b + 2) * NW, i2, (b + 2) % 4)

        @pl.when(b < BPW - 1)
        def _():
            wait_idx(i2n)
            issue_gather(i2n)

        @plsc.parallel_loop(0, EB, step=1, unroll=8)
        def _(r):
            for j in range(D // 32):
                x32 = plsc.bitcast(rows_v[i2, r, pl.ds(j * LANES, LANES)],
                                   jnp.bfloat16)
                xa, xb = plsc.unpack(x32, format=plsc.PackFormat.INTERLEAVED)
                sla = pl.ds(j * 32, LANES)
                slb = pl.ds(j * 32 + LANES, LANES)
                m_v[e3, r, sla] = jnp.maximum(xa + m_v[e3, r, sla], 0.0)
                m_v[e3, r, slb] = jnp.maximum(xb + m_v[e3, r, slb], 0.0)

        pltpu.async_copy(m_v.at[e3], agg_sh.at[dsti.at[b % 4]], sem_s.at[e3], add=True)
        return 0
    lax.fori_loop(0, BPW, edge_block, 0)

    wait_scatter((BPW - 2) % 3, (BPW - 2) % 4)
    wait_scatter((BPW - 1) % 3, (BPW - 1) % 4)
    plsc.subcore_barrier()

    for k in range(5):
        r0 = (sid * 5 + k) * 128
        pltpu.sync_copy(agg_sh.at[pl.ds(r0, 128)], out_hbm.at[cid, pl.ds(r0, 128)])


_sc_edge = pl.kernel(
    _sc_edge_body,
    out_type=jax.ShapeDtypeStruct((2, NPAD, D), jnp.float32),
    mesh=plsc.VectorSubcoreMesh(core_axis_name="c", subcore_axis_name="s",
                                num_cores=2, num_subcores=16),
    compiler_params=pltpu.CompilerParams(use_tc_tiling_on_sc=False,
                                         needs_layout_passes=False),
    scratch_types=[
        pltpu.VMEM((2, EB, D // 2), jnp.float32),
        pltpu.VMEM((3, EB, D), jnp.float32),
        pltpu.VMEM((2, EB), jnp.int32),
        pltpu.VMEM((4, EB), jnp.int32),
        pltpu.VMEM_SHARED((NPAD, D), jnp.float32),
        pltpu.SemaphoreType.DMA((2,)),
        pltpu.SemaphoreType.DMA((2,)),
        pltpu.SemaphoreType.DMA((2,)),
        pltpu.SemaphoreType.DMA((3,)),
        pltpu.SemaphoreType.DMA((3,)),
    ],
)


def _tc_dense_body(x_ref, agg_ref, seg_ref, W1_ref, b1_ref, W2_ref, b2_ref,
                   eps_ref, Wl_ref, bl_ref, out_ref, h_ref, pooled_ref):
    h0 = (1.0 + eps_ref[...]) * x_ref[...] + agg_ref[0, :N, :] + agg_ref[1, :N, :]
    t = jnp.maximum(
        jnp.dot(h0.astype(jnp.bfloat16), W1_ref[...],
                preferred_element_type=jnp.float32) + b1_ref[...],
        0.0)
    h_ref[...] = jnp.dot(t.astype(jnp.bfloat16), W2_ref[...],
                         preferred_element_type=jnp.float32) + b2_ref[...]

    def pool_g(g, _):
        s0 = seg_ref[g]
        s1 = seg_ref[g + 1]
        c0 = s0 // 8
        c1 = jnp.maximum((s1 - 1) // 8, 0)

        def masked(c, acc):
            off = c * 8
            blockv = h_ref[pl.ds(off, 8), :]
            rowid = off + lax.broadcasted_iota(jnp.int32, (8, HID), 0)
            m = (rowid >= s0) & (rowid < s1)
            return jnp.maximum(acc, jnp.where(m, blockv, -3.4e38))

        acc = masked(c0, jnp.full((8, HID), -3.4e38, jnp.float32))
        acc = masked(c1, acc)

        ni = jnp.maximum(c1 - c0 - 1, 0)
        n4 = ni // 4

        def interior4(j, acc):
            off = (c0 + 1 + j * 4) * 8
            blk = h_ref[pl.ds(off, 32), :].reshape(4, 8, HID)
            return jnp.maximum(acc, jnp.max(blk, axis=0))
        acc = lax.fori_loop(0, n4, interior4, acc)

        def interior1(j, acc):
            return jnp.maximum(acc, h_ref[pl.ds((c0 + 1 + n4 * 4 + j) * 8, 8), :])
        acc = lax.fori_loop(0, ni - n4 * 4, interior1, acc)

        pooled_ref[pl.ds(g, 1), :] = jnp.max(acc, axis=0, keepdims=True)
        return 0
    lax.fori_loop(0, G, pool_g, 0)

    out_ref[...] = (jnp.dot(pooled_ref[...], Wl_ref[...],
                            preferred_element_type=jnp.float32)
                    + bl_ref[...])


_tc_dense = pl.pallas_call(
    _tc_dense_body,
    out_shape=jax.ShapeDtypeStruct((G, OUT), jnp.float32),
    in_specs=[pl.BlockSpec(memory_space=pltpu.VMEM)] * 2
             + [pl.BlockSpec(memory_space=pltpu.SMEM)]
             + [pl.BlockSpec(memory_space=pltpu.VMEM)] * 7,
    scratch_shapes=[pltpu.VMEM((N, HID), jnp.float32),
                    pltpu.VMEM((G, HID), jnp.float32)],
)


def kernel(x, edge_index, edge_weights, batch, W1, b1, W2, b2, eps, Wl, bl):
    src = edge_index[0]
    dst = edge_index[1]
    seg = jnp.searchsorted(batch, jnp.arange(G + 1, dtype=jnp.int32)).astype(jnp.int32)
    xq = (x.reshape(N, D // 32, 2, 16).swapaxes(2, 3)
          .reshape(N, D).astype(jnp.bfloat16))
    xq32 = lax.bitcast_convert_type(xq.reshape(N, D // 2, 2), jnp.float32)
    agg = _sc_edge(xq32, src, dst, edge_weights)
    out = _tc_dense(x, agg, seg,
                    W1.astype(jnp.bfloat16), b1.reshape(1, -1),
                    W2.astype(jnp.bfloat16), b2.reshape(1, -1),
                    eps.reshape(1, 1), Wl, bl.reshape(1, -1))
    return out
